# baseline (speedup 1.0000x reference)
import functools

import jax
import jax.numpy as jnp
from jax import lax
from jax.experimental import pallas as pl
from jax.experimental.pallas import tpu as pltpu
from jax.experimental.pallas import tpu_sc as plsc

T = 100
TPAD = 128
NV = TPAD // 32
KTOP = 2
B, D, C, P = 32, 2, 3, 2048
NEG_DUP = 0xC000C000


def _rne_bf16_bits(f):
    u = plsc.bitcast(f, jnp.uint32)
    return (u + jnp.uint32(0x7FFF) + ((u >> jnp.uint32(16)) & jnp.uint32(1))) >> jnp.uint32(16)


def _pl_sc_call(pd_flat):
    mesh = plsc.VectorSubcoreMesh(core_axis_name="c", subcore_axis_name="s")

    slab = C * P * 2
    oslab = KTOP * C * (TPAD // 2)

    @functools.partial(
        pl.kernel,
        mesh=mesh,
        compiler_params=pltpu.CompilerParams(needs_layout_passes=False),
        out_type=jax.ShapeDtypeStruct((B * D * oslab,), jnp.uint32),
        scratch_types=[
            pltpu.VMEM((slab,), jnp.float32),
            pltpu.VMEM((oslab,), jnp.uint32),
        ],
    )
    def sc_kernel(pd_hbm, out_hbm, in_v, out_v):
        wid = lax.axis_index("s") * 2 + lax.axis_index("c")
        lane2 = lax.iota(jnp.int32, 16).astype(jnp.float32) * 2.0
        tvecs = []
        for j in range(NV):
            te = (lane2 + float(32 * j)) * (1.0 / (T - 1))
            to = (lane2 + float(32 * j + 1)) * (1.0 / (T - 1))
            tu = _rne_bf16_bits(te) | (_rne_bf16_bits(to) << jnp.uint32(16))
            tvecs.append(plsc.bitcast(tu, jnp.bfloat16))

        def dup16(w):
            r = _rne_bf16_bits(w)
            return r | (r << jnp.uint32(16))

        def bcast16(w, i):
            return plsc.bitcast(jnp.full((16,), w[i], jnp.uint32), jnp.bfloat16)

        def update(m1, m2, w, i):
            bv = bcast16(w, 2 * i)
            dv = bcast16(w, 2 * i + 1)
            nm1, nm2 = [], []
            for j in range(NV):
                v = jnp.minimum(tvecs[j] - bv, dv - tvecs[j])
                nm2.append(jnp.maximum(m2[j], jnp.minimum(m1[j], v)))
                nm1.append(jnp.maximum(m1[j], v))
            return tuple(nm1), tuple(nm2)

        neg = plsc.bitcast(jnp.full((16,), NEG_DUP, jnp.uint32), jnp.bfloat16)
        zero = plsc.bitcast(jnp.full((16,), 0, jnp.uint32), jnp.bfloat16)

        for d in range(D):
            pltpu.sync_copy(pd_hbm.at[pl.ds((wid * D + d) * slab, slab)], in_v)
            nbars = P - 1 if d == 0 else P
            nfull = nbars // 16
            ntail = nbars % 16

            for c in range(C):
                base = c * P * 2

                def body(k, carry, base=base):
                    m1, m2 = carry
                    w1 = dup16(in_v[pl.ds(base + 32 * k, 16)])
                    w2 = dup16(in_v[pl.ds(base + 32 * k + 16, 16)])
                    for i in range(8):
                        m1, m2 = update(m1, m2, w1, i)
                    for i in range(8):
                        m1, m2 = update(m1, m2, w2, i)
                    return m1, m2

                init = (tuple(neg for _ in range(NV)),
                        tuple(neg for _ in range(NV)))
                m1, m2 = lax.fori_loop(0, nfull, body, init)
                if ntail:
                    w1 = dup16(in_v[pl.ds(base + 32 * nfull, 16)])
                    w2 = dup16(in_v[pl.ds(base + 32 * nfull + 16, 16)])
                    for ti in range(min(ntail, 8)):
                        m1, m2 = update(m1, m2, w1, ti)
                    for ti in range(max(ntail - 8, 0)):
                        m1, m2 = update(m1, m2, w2, ti)

                for j in range(NV):
                    off = c * (TPAD // 2) + 16 * j
                    out_v[pl.ds(off, 16)] = plsc.bitcast(
                        jnp.maximum(m1[j], zero), jnp.uint32)
                    out_v[pl.ds(C * (TPAD // 2) + off, 16)] = plsc.bitcast(
                        jnp.maximum(m2[j], zero), jnp.uint32)

            pltpu.sync_copy(
                out_v, out_hbm.at[pl.ds((wid * D + d) * oslab, oslab)])

    return sc_kernel(pd_flat)


@jax.jit
def kernel(pd):
    out_u = _pl_sc_call(pd.reshape(B * D * C * P * 2)).reshape(
        B, D, KTOP, C, TPAD // 2)
    out_bf = lax.bitcast_convert_type(
        lax.bitcast_convert_type(out_u, jnp.uint16), jnp.bfloat16)
    out = out_bf.reshape(B, D, KTOP, C, TPAD).astype(jnp.float32)
    return out[..., :T].reshape(B, D, KTOP, C * T)


# speedup vs baseline: 2.6599x; 2.6599x over previous
import functools

import jax
import jax.numpy as jnp
from jax import lax
from jax.experimental import pallas as pl
from jax.experimental.pallas import tpu as pltpu
from jax.experimental.pallas import tpu_sc as plsc

T = 100
TPAD = 128
NV = TPAD // 32
KTOP = 2
B, D, C, P = 32, 2, 3, 2048
NEG_DUP = 0xC000C000


def _rne_bf16_bits(f):
    u = plsc.bitcast(f, jnp.uint32)
    return (u + jnp.uint32(0x7FFF) + ((u >> jnp.uint32(16)) & jnp.uint32(1))) >> jnp.uint32(16)


def _pl_sc_call(pd_flat):
    mesh = plsc.VectorSubcoreMesh(core_axis_name="c", subcore_axis_name="s")

    slab = C * P * 2
    oslab = KTOP * C * (TPAD // 2)

    @functools.partial(
        pl.kernel,
        mesh=mesh,
        compiler_params=pltpu.CompilerParams(needs_layout_passes=False),
        out_type=jax.ShapeDtypeStruct((B * D * oslab,), jnp.uint32),
        scratch_types=[
            pltpu.VMEM((slab,), jnp.float32),
            pltpu.VMEM((oslab,), jnp.uint32),
        ],
    )
    def sc_kernel(pd_hbm, out_hbm, in_v, out_v):
        wid = lax.axis_index("s") * 2 + lax.axis_index("c")
        lane2 = lax.iota(jnp.int32, 16).astype(jnp.float32) * 2.0
        tvecs = []
        for j in range(NV):
            te = (lane2 + float(32 * j)) * (1.0 / (T - 1))
            to = (lane2 + float(32 * j + 1)) * (1.0 / (T - 1))
            tu = _rne_bf16_bits(te) | (_rne_bf16_bits(to) << jnp.uint32(16))
            tvecs.append(plsc.bitcast(tu, jnp.bfloat16))

        def dup16(w):
            r = _rne_bf16_bits(w)
            return r | (r << jnp.uint32(16))

        def bcast16(w, i):
            return plsc.bitcast(jnp.full((16,), w[i], jnp.uint32), jnp.bfloat16)

        def update(m1, m2, w, i):
            bv = bcast16(w, 2 * i)
            dv = bcast16(w, 2 * i + 1)
            nm1, nm2 = [], []
            for j in range(NV):
                v = jnp.minimum(tvecs[j] - bv, dv - tvecs[j])
                nm2.append(jnp.maximum(m2[j], jnp.minimum(m1[j], v)))
                nm1.append(jnp.maximum(m1[j], v))
            return tuple(nm1), tuple(nm2)

        neg = plsc.bitcast(jnp.full((16,), NEG_DUP, jnp.uint32), jnp.bfloat16)
        zero = plsc.bitcast(jnp.full((16,), 0, jnp.uint32), jnp.bfloat16)

        for d in range(D):
            pltpu.sync_copy(pd_hbm.at[wid, d], in_v)
            nbars = P - 1 if d == 0 else P
            nfull = nbars // 16
            ntail = nbars % 16

            for c in range(C):
                base = c * P * 2

                def body(k, carry, base=base):
                    m1, m2 = carry
                    w1 = dup16(in_v[pl.ds(base + 32 * k, 16)])
                    w2 = dup16(in_v[pl.ds(base + 32 * k + 16, 16)])
                    for i in range(8):
                        m1, m2 = update(m1, m2, w1, i)
                    for i in range(8):
                        m1, m2 = update(m1, m2, w2, i)
                    return m1, m2

                init = (tuple(neg for _ in range(NV)),
                        tuple(neg for _ in range(NV)))
                m1, m2 = lax.fori_loop(0, nfull, body, init)
                if ntail:
                    w1 = dup16(in_v[pl.ds(base + 32 * nfull, 16)])
                    w2 = dup16(in_v[pl.ds(base + 32 * nfull + 16, 16)])
                    for ti in range(min(ntail, 8)):
                        m1, m2 = update(m1, m2, w1, ti)
                    for ti in range(max(ntail - 8, 0)):
                        m1, m2 = update(m1, m2, w2, ti)

                for j in range(NV):
                    off = c * (TPAD // 2) + 16 * j
                    out_v[pl.ds(off, 16)] = plsc.bitcast(
                        jnp.maximum(m1[j], zero), jnp.uint32)
                    out_v[pl.ds(C * (TPAD // 2) + off, 16)] = plsc.bitcast(
                        jnp.maximum(m2[j], zero), jnp.uint32)

            pltpu.sync_copy(
                out_v, out_hbm.at[pl.ds((wid * D + d) * oslab, oslab)])

    return sc_kernel(pd_flat)


@jax.jit
def kernel(pd):
    out_u = _pl_sc_call(pd.reshape(B, D, C * P * 2)).reshape(
        B, D, KTOP, C, TPAD // 2)
    out_bf = lax.bitcast_convert_type(
        lax.bitcast_convert_type(out_u, jnp.uint16), jnp.bfloat16)
    out = out_bf.reshape(B, D, KTOP, C, TPAD).astype(jnp.float32)
    return out[..., :T].reshape(B, D, KTOP, C * T)
